# baseline (speedup 1.0000x reference)
import math

import jax
import jax.numpy as jnp
from jax.experimental import pallas as pl
from jax.experimental.pallas import tpu as pltpu


def _round_up(x: int, m: int) -> int:
    return ((x + m - 1) // m) * m


def _gram_body(x1_ref, x2_ref, out_ref):
    a = x1_ref[0].astype(jnp.bfloat16)
    b = x2_ref[0].astype(jnp.bfloat16)
    out_ref[0] = jax.lax.dot_general(
        a,
        b,
        dimension_numbers=(((1,), (1,)), ((), ())),
        preferred_element_type=jnp.float32,
    )


def kernel(X1: jax.Array, X2: jax.Array) -> jax.Array:
    if X1.shape[-1] != X2.shape[-1]:
        raise ValueError(
            f"Input vectors must have the same feature dimension. "
            f"Got X1 dim {X1.shape[-1]} and X2 dim {X2.shape[-1]}"
        )

    N, D = X1.shape[-2], X1.shape[-1]
    M = X2.shape[-2]
    batch_shape = jnp.broadcast_shapes(X1.shape[:-2], X2.shape[:-2])
    B = math.prod(batch_shape) if batch_shape else 1

    x1 = jnp.broadcast_to(X1.astype(jnp.float32), (*batch_shape, N, D))
    x2 = jnp.broadcast_to(X2.astype(jnp.float32), (*batch_shape, M, D))
    x1 = x1.reshape(B, N, D)
    x2 = x2.reshape(B, M, D)

    N_pad = _round_up(N, 8)
    M_pad = _round_up(M, 128)
    D_pad = _round_up(D, 128)

    tn = N_pad
    while tn > 256 and tn % 2 == 0 and (tn // 2) % 8 == 0:
        tn = tn // 2
    N_pad = _round_up(N_pad, tn)

    def _pad(x, rows, rows_pad):
        pads = ((0, 0), (0, rows_pad - rows), (0, D_pad - D))
        return jnp.pad(x, pads) if any(p[1] for p in pads) else x

    x1p = _pad(x1, N, N_pad)
    x2p = _pad(x2, M, M_pad)

    grid = (B, N_pad // tn)

    block_bytes = (tn * D_pad + M_pad * D_pad) * 4 + tn * M_pad * 4
    vmem_limit = int(min(60 * 1024 * 1024, max(16 * 1024 * 1024, 3 * block_bytes)))

    cost = pl.CostEstimate(
        flops=2 * B * N_pad * M_pad * D_pad,
        transcendentals=0,
        bytes_accessed=4 * B * (N_pad + M_pad) * D_pad + 4 * B * N_pad * M_pad,
    )

    out = pl.pallas_call(
        _gram_body,
        out_shape=jax.ShapeDtypeStruct((B, N_pad, M_pad), jnp.float32),
        grid_spec=pltpu.PrefetchScalarGridSpec(
            num_scalar_prefetch=0,
            grid=grid,
            in_specs=[
                pl.BlockSpec((1, tn, D_pad), lambda b, i: (b, i, 0)),
                pl.BlockSpec((1, M_pad, D_pad), lambda b, i: (b, 0, 0)),
            ],
            out_specs=pl.BlockSpec((1, tn, M_pad), lambda b, i: (b, i, 0)),
        ),
        compiler_params=pltpu.CompilerParams(
            dimension_semantics=("parallel", "arbitrary"),
            vmem_limit_bytes=vmem_limit,
        ),
        cost_estimate=cost,
    )(x1p, x2p)

    out = out[:, :N, :M]
    return out.reshape(*batch_shape, N, M)


# speedup vs baseline: 1.2367x; 1.2367x over previous
import math

import jax
import jax.numpy as jnp
from jax.experimental import pallas as pl
from jax.experimental.pallas import tpu as pltpu


def _round_up(x: int, m: int) -> int:
    return ((x + m - 1) // m) * m


def _gram_body(x1_ref, x2_ref, out_ref):
    a = x1_ref[0].astype(jnp.bfloat16)
    b = x2_ref[0].astype(jnp.bfloat16)
    out_ref[0] = jax.lax.dot_general(
        a,
        b,
        dimension_numbers=(((1,), (1,)), ((), ())),
        preferred_element_type=jnp.float32,
    )


def kernel(X1: jax.Array, X2: jax.Array) -> jax.Array:
    if X1.shape[-1] != X2.shape[-1]:
        raise ValueError(
            f"Input vectors must have the same feature dimension. "
            f"Got X1 dim {X1.shape[-1]} and X2 dim {X2.shape[-1]}"
        )

    N, D = X1.shape[-2], X1.shape[-1]
    M = X2.shape[-2]
    batch_shape = jnp.broadcast_shapes(X1.shape[:-2], X2.shape[:-2])
    B = math.prod(batch_shape) if batch_shape else 1

    x1 = jnp.broadcast_to(X1.astype(jnp.float32), (*batch_shape, N, D))
    x2 = jnp.broadcast_to(X2.astype(jnp.float32), (*batch_shape, M, D))
    x1 = x1.reshape(B, N, D)
    x2 = x2.reshape(B, M, D)

    N_pad = _round_up(N, 8)
    M_pad = _round_up(M, 128)
    D_pad = _round_up(D, 128)

    tn = N_pad
    while tn > 512 and tn % 2 == 0 and (tn // 2) % 8 == 0:
        tn = tn // 2
    N_pad = _round_up(N_pad, tn)

    def _pad(x, rows, rows_pad):
        pads = ((0, 0), (0, rows_pad - rows), (0, D_pad - D))
        return jnp.pad(x, pads) if any(p[1] for p in pads) else x

    x1p = _pad(x1, N, N_pad)
    x2p = _pad(x2, M, M_pad)

    grid = (B, N_pad // tn)

    block_bytes = (tn * D_pad + M_pad * D_pad) * 4 + tn * M_pad * 4
    vmem_limit = int(min(60 * 1024 * 1024, max(16 * 1024 * 1024, 3 * block_bytes)))

    cost = pl.CostEstimate(
        flops=2 * B * N_pad * M_pad * D_pad,
        transcendentals=0,
        bytes_accessed=4 * B * (N_pad + M_pad) * D_pad + 4 * B * N_pad * M_pad,
    )

    out = pl.pallas_call(
        _gram_body,
        out_shape=jax.ShapeDtypeStruct((B, N_pad, M_pad), jnp.float32),
        grid_spec=pltpu.PrefetchScalarGridSpec(
            num_scalar_prefetch=0,
            grid=grid,
            in_specs=[
                pl.BlockSpec((1, tn, D_pad), lambda b, i: (b, i, 0)),
                pl.BlockSpec((1, M_pad, D_pad), lambda b, i: (b, 0, 0)),
            ],
            out_specs=pl.BlockSpec((1, tn, M_pad), lambda b, i: (b, i, 0)),
        ),
        compiler_params=pltpu.CompilerParams(
            dimension_semantics=("parallel", "arbitrary"),
            vmem_limit_bytes=vmem_limit,
        ),
        cost_estimate=cost,
    )(x1p, x2p)

    out = out[:, :N, :M]
    return out.reshape(*batch_shape, N, M)


# speedup vs baseline: 1.5113x; 1.2221x over previous
import math

import jax
import jax.numpy as jnp
from jax.experimental import pallas as pl
from jax.experimental.pallas import tpu as pltpu


def _round_up(x: int, m: int) -> int:
    return ((x + m - 1) // m) * m


def _gram_body(x1_ref, x2_ref, out_ref):
    a = x1_ref[...].astype(jnp.bfloat16)
    b = x2_ref[...].astype(jnp.bfloat16)
    out_ref[...] = jax.lax.dot_general(
        a,
        b,
        dimension_numbers=(((2,), (2,)), ((0,), (0,))),
        preferred_element_type=jnp.float32,
    )


def kernel(X1: jax.Array, X2: jax.Array) -> jax.Array:
    if X1.shape[-1] != X2.shape[-1]:
        raise ValueError(
            f"Input vectors must have the same feature dimension. "
            f"Got X1 dim {X1.shape[-1]} and X2 dim {X2.shape[-1]}"
        )

    N, D = X1.shape[-2], X1.shape[-1]
    M = X2.shape[-2]
    batch_shape = jnp.broadcast_shapes(X1.shape[:-2], X2.shape[:-2])
    B = math.prod(batch_shape) if batch_shape else 1

    x1 = jnp.broadcast_to(X1.astype(jnp.float32), (*batch_shape, N, D))
    x2 = jnp.broadcast_to(X2.astype(jnp.float32), (*batch_shape, M, D))
    x1 = x1.reshape(B, N, D)
    x2 = x2.reshape(B, M, D)

    N_pad = _round_up(N, 8)
    M_pad = _round_up(M, 128)
    D_pad = _round_up(D, 128)

    per_batch_bytes = (N_pad * D_pad + M_pad * D_pad) * 4 + N_pad * M_pad * 4
    bt = max(1, min(B, (20 * 1024 * 1024) // max(per_batch_bytes, 1)))
    if B > 1:
        bt = min(bt, B // 2 if B % 2 == 0 else max(1, -(-B // 2)))
    while B % bt:
        bt -= 1
    steps = B // bt

    def _pad(x, rows, rows_pad):
        pads = ((0, 0), (0, rows_pad - rows), (0, D_pad - D))
        return jnp.pad(x, pads) if any(p[1] for p in pads) else x

    x1p = _pad(x1, N, N_pad)
    x2p = _pad(x2, M, M_pad)

    block_bytes = bt * per_batch_bytes
    vmem_limit = int(min(60 * 1024 * 1024, max(16 * 1024 * 1024, 3 * block_bytes)))

    cost = pl.CostEstimate(
        flops=2 * B * N_pad * M_pad * D_pad,
        transcendentals=0,
        bytes_accessed=4 * B * (N_pad + M_pad) * D_pad + 4 * B * N_pad * M_pad,
    )

    out = pl.pallas_call(
        _gram_body,
        out_shape=jax.ShapeDtypeStruct((B, N_pad, M_pad), jnp.float32),
        grid_spec=pltpu.PrefetchScalarGridSpec(
            num_scalar_prefetch=0,
            grid=(steps,),
            in_specs=[
                pl.BlockSpec((bt, N_pad, D_pad), lambda i: (i, 0, 0)),
                pl.BlockSpec((bt, M_pad, D_pad), lambda i: (i, 0, 0)),
            ],
            out_specs=pl.BlockSpec((bt, N_pad, M_pad), lambda i: (i, 0, 0)),
        ),
        compiler_params=pltpu.CompilerParams(
            dimension_semantics=("parallel",),
            vmem_limit_bytes=vmem_limit,
        ),
        cost_estimate=cost,
    )(x1p, x2p)

    out = out[:, :N, :M]
    return out.reshape(*batch_shape, N, M)
